# baseline (speedup 1.0000x reference)
import jax
import jax.numpy as jnp
from jax.experimental import pallas as pl

N = 10000
E = 320000
D = 128
H = 8
DH = D // H
R = 50
CUT_UP = 5.0
EPS = 1e-5

_NB = 1000
_EB = 4000


def _silu(x):
    return x * jax.nn.sigmoid(x)


def _node_body(x_ref, vec_ref, ln_w_ref, ln_b_ref, Wq_ref, bq_ref, Wk_ref, bk_ref,
               Wv_ref, bv_ref, Wvec_ref, bvec_ref,
               q_ref, k_ref, v_ref, vdot_ref, vec3_ref):
    x = x_ref[...]
    mu = jnp.mean(x, axis=-1, keepdims=True)
    var = jnp.mean((x - mu) ** 2, axis=-1, keepdims=True)
    xn = (x - mu) / jnp.sqrt(var + EPS) * ln_w_ref[...] + ln_b_ref[...]
    q_ref[...] = xn @ Wq_ref[...] + bq_ref[...]
    k_ref[...] = xn @ Wk_ref[...] + bk_ref[...]
    v_ref[...] = xn @ Wv_ref[...] + bv_ref[...]
    Wvec = Wvec_ref[...]
    bvec = bvec_ref[...]
    vdot = jnp.zeros((_NB, D), jnp.float32)
    for i in range(3):
        vp = vec_ref[:, i, :] @ Wvec + bvec
        vdot = vdot + vp[:, :D] * vp[:, D:2 * D]
        vec3_ref[:, i, :] = vp[:, 2 * D:]
    vdot_ref[...] = vdot


def _edge_body(f_ref, r_ref, Wdk_ref, bdk_ref, Wdv_ref, bdv_ref,
               dk_ref, dv_ref, cut_ref):
    f = f_ref[...]
    dk_ref[...] = _silu(f @ Wdk_ref[...] + bdk_ref[...])
    dv_ref[...] = _silu(f @ Wdv_ref[...] + bdv_ref[...])
    r = r_ref[...]
    cut_ref[...] = 0.5 * (jnp.cos(r * jnp.pi / CUT_UP) + 1.0) * (r < CUT_UP).astype(r.dtype)


def _final_body(xagg_ref, vagg_ref, vdot_ref, vec3_ref, Wo_ref, bo_ref,
                dx_ref, dvec_ref):
    o = xagg_ref[...] @ Wo_ref[...] + bo_ref[...]
    o1 = o[:, :D]
    o2 = o[:, D:2 * D]
    o3 = o[:, 2 * D:]
    dx_ref[...] = vdot_ref[...] * o2 + o3
    dvec_ref[...] = vec3_ref[...] * o1[:, None, :] + vagg_ref[...]


def kernel(x, vec, edge_index, r_ij, f_ij, d_ij, ln_w, ln_b, Wq, bq, Wk, bk,
           Wv, bv, Wo, bo, Wvec, bvec, Wdk, bdk, Wdv, bdv):
    nw = lambda shp: pl.BlockSpec(shp, lambda i: (0,) * len(shp))
    q, k, v, vdot, vec3 = pl.pallas_call(
        _node_body,
        grid=(N // _NB,),
        in_specs=[
            pl.BlockSpec((_NB, D), lambda i: (i, 0)),
            pl.BlockSpec((_NB, 3, D), lambda i: (i, 0, 0)),
            nw((D,)), nw((D,)),
            nw((D, D)), nw((D,)),
            nw((D, D)), nw((D,)),
            nw((D, 3 * D)), nw((3 * D,)),
            nw((D, 3 * D)), nw((3 * D,)),
        ],
        out_specs=[
            pl.BlockSpec((_NB, D), lambda i: (i, 0)),
            pl.BlockSpec((_NB, D), lambda i: (i, 0)),
            pl.BlockSpec((_NB, 3 * D), lambda i: (i, 0)),
            pl.BlockSpec((_NB, D), lambda i: (i, 0)),
            pl.BlockSpec((_NB, 3, D), lambda i: (i, 0, 0)),
        ],
        out_shape=[
            jax.ShapeDtypeStruct((N, D), jnp.float32),
            jax.ShapeDtypeStruct((N, D), jnp.float32),
            jax.ShapeDtypeStruct((N, 3 * D), jnp.float32),
            jax.ShapeDtypeStruct((N, D), jnp.float32),
            jax.ShapeDtypeStruct((N, 3, D), jnp.float32),
        ],
    )(x, vec, ln_w, ln_b, Wq, bq, Wk, bk, Wv, bv, Wvec, bvec)

    dk, dv, cut = pl.pallas_call(
        _edge_body,
        grid=(E // _EB,),
        in_specs=[
            pl.BlockSpec((_EB, R), lambda i: (i, 0)),
            pl.BlockSpec((_EB, 1), lambda i: (i, 0)),
            nw((R, D)), nw((D,)),
            nw((R, 3 * D)), nw((3 * D,)),
        ],
        out_specs=[
            pl.BlockSpec((_EB, D), lambda i: (i, 0)),
            pl.BlockSpec((_EB, 3 * D), lambda i: (i, 0)),
            pl.BlockSpec((_EB, 1), lambda i: (i, 0)),
        ],
        out_shape=[
            jax.ShapeDtypeStruct((E, D), jnp.float32),
            jax.ShapeDtypeStruct((E, 3 * D), jnp.float32),
            jax.ShapeDtypeStruct((E, 1), jnp.float32),
        ],
    )(f_ij, r_ij.reshape(E, 1), Wdk, bdk, Wdv, bdv)

    src = edge_index[0]
    dst = edge_index[1]
    qh = q.reshape(N, H, DH)
    kh = k.reshape(N, H, DH)
    vh = v.reshape(N, H, 3 * DH)
    dkh = dk.reshape(E, H, DH)
    dvh = dv.reshape(E, H, 3 * DH)
    attn = _silu(jnp.sum(qh[dst] * kh[src] * dkh, axis=-1)) * cut
    vj = vh[src] * dvh
    xm = vj[..., :DH] * attn[..., None]
    v1m = vj[..., DH:2 * DH]
    v2m = vj[..., 2 * DH:]
    vec_r = vec.reshape(N, 3, H, DH)
    vecm = vec_r[src] * v1m[:, None] + v2m[:, None] * d_ij[:, :, None, None]
    x_agg = jax.ops.segment_sum(xm, dst, num_segments=N).reshape(N, D)
    vec_agg = jax.ops.segment_sum(vecm, dst, num_segments=N).reshape(N, 3, D)

    dx, dvec = pl.pallas_call(
        _final_body,
        grid=(N // _NB,),
        in_specs=[
            pl.BlockSpec((_NB, D), lambda i: (i, 0)),
            pl.BlockSpec((_NB, 3, D), lambda i: (i, 0, 0)),
            pl.BlockSpec((_NB, D), lambda i: (i, 0)),
            pl.BlockSpec((_NB, 3, D), lambda i: (i, 0, 0)),
            nw((D, 3 * D)), nw((3 * D,)),
        ],
        out_specs=[
            pl.BlockSpec((_NB, D), lambda i: (i, 0)),
            pl.BlockSpec((_NB, 3, D), lambda i: (i, 0, 0)),
        ],
        out_shape=[
            jax.ShapeDtypeStruct((N, D), jnp.float32),
            jax.ShapeDtypeStruct((N, 3, D), jnp.float32),
        ],
    )(x_agg, vec_agg, vdot, vec3, Wo, bo)
    return (dx, dvec)


# speedup vs baseline: 6.6550x; 6.6550x over previous
import jax
import jax.numpy as jnp
from jax import lax
from jax.experimental import pallas as pl
from jax.experimental.pallas import tpu as pltpu
from jax.experimental.pallas import tpu_sc as plsc

N = 10000
E = 320000
D = 128
H = 8
DH = D // H
R = 50
CUT_UP = 5.0
EPS = 1e-5

NC = 2
NS = 16
NW = NC * NS

B = 16
NBLK = E // B
JMAX = -(-NBLK // NW)
NPAD = 10240
ROWS = NPAD // NS
ZR = 128
CW = 64
SPG = 7

_NB = 1000
_EB = 4000


def _silu(x):
    return x * (1.0 / (1.0 + jnp.exp(-x)))


def _node_body(x_ref, vec_ref, ln_w_ref, ln_b_ref, Wq_ref, bq_ref, Wk_ref,
               bk_ref, Wv_ref, bv_ref, Wvec_ref, bvec_ref,
               q_ref, t_ref, vdot_ref, vec3_ref):
    x = x_ref[...]
    mu = jnp.mean(x, axis=-1, keepdims=True)
    var = jnp.mean((x - mu) ** 2, axis=-1, keepdims=True)
    xn = (x - mu) / jnp.sqrt(var + EPS) * ln_w_ref[...] + ln_b_ref[...]
    q_ref[...] = xn @ Wq_ref[...] + bq_ref[...]
    t_ref[:, 0:D] = xn @ Wk_ref[...] + bk_ref[...]
    t_ref[:, D:4 * D] = xn @ Wv_ref[...] + bv_ref[...]
    Wvec = Wvec_ref[...]
    bvec = bvec_ref[...]
    vdot = jnp.zeros((_NB, D), jnp.float32)
    for i in range(3):
        vi = vec_ref[:, i, :]
        t_ref[:, 4 * D + i * D:4 * D + (i + 1) * D] = vi
        vp = vi @ Wvec + bvec
        vdot = vdot + vp[:, :D] * vp[:, D:2 * D]
        vec3_ref[:, i, :] = vp[:, 2 * D:]
    vdot_ref[...] = vdot


def _edge_body(f_ref, r_ref, d_ref, Wdk_ref, bdk_ref, Wdv_ref, bdv_ref,
               ef_ref, mt_ref):
    f = f_ref[...]
    ef_ref[:, 0:D] = _silu(f @ Wdk_ref[...] + bdk_ref[...])
    ef_ref[:, D:4 * D] = _silu(f @ Wdv_ref[...] + bdv_ref[...])
    r = r_ref[...]
    cut = 0.5 * (jnp.cos(r * jnp.pi / CUT_UP) + 1.0) * (r < CUT_UP).astype(r.dtype)
    col = lax.broadcasted_iota(jnp.int32, (_EB, 64), 1) // 16
    d = d_ref[...]
    m = cut * (col == 0).astype(jnp.float32)
    for i in range(3):
        m = m + d[:, i:i + 1] * (col == i + 1).astype(jnp.float32)
    mt_ref[...] = m


def _sc_body(src_hbm, dst_hbm, q_hbm, t_hbm, ef_hbm, mt_hbm,
             px_hbm, pv_hbm, msg_hbm,
             sid, did, qb, tb, eb, mb, xlo, sp, zb, acc, sem):
    c = lax.axis_index("c")
    s = lax.axis_index("s")
    wid = s * NC + c

    def _zrow(i, _):
        for l in range(CW // 16):
            zb[i, pl.ds(l * 16, 16)] = jnp.zeros((16,), jnp.float32)
        return 0
    lax.fori_loop(0, ZR, _zrow, 0)

    def _zero_acc():
        for rep in range(ROWS // ZR):
            pltpu.sync_copy(zb, acc.at[pl.ds(s * ROWS + rep * ZR, ZR)])

    _zero_acc()
    plsc.subcore_barrier()

    def blockA(j, _):
        g = j * NW + wid

        @pl.when(g < NBLK)
        def _():
            base = g * B
            pltpu.sync_copy(src_hbm.at[pl.ds(base, B)], sid)
            pltpu.sync_copy(dst_hbm.at[pl.ds(base, B)], did)
            pltpu.async_copy(t_hbm.at[sid], tb, sem).wait()
            pltpu.async_copy(q_hbm.at[did], qb, sem).wait()
            pltpu.sync_copy(ef_hbm.at[pl.ds(base, B), :], eb)
            pltpu.sync_copy(mt_hbm.at[pl.ds(base, B), :], mb)

            def edge(e, _):
                cutv = mb[e, pl.ds(0, 16)]
                dvs = tuple(mb[e, pl.ds(16 * (i + 1), 16)] for i in range(3))
                for h in range(H):
                    t = (qb[e, pl.ds(h * 16, 16)]
                         * tb[e, pl.ds(h * 16, 16)]
                         * eb[e, pl.ds(h * 16, 16)])
                    sv = jnp.broadcast_to(jnp.sum(t), (16,))
                    av = sv * (1.0 / (1.0 + jnp.exp(-sv))) * cutv
                    vo = D + h * 48
                    vx = tb[e, pl.ds(vo, 16)] * eb[e, pl.ds(vo, 16)]
                    if h < 4:
                        xlo[e, pl.ds(h * 16, 16)] = vx * av
                    else:
                        sp[0, e, pl.ds((h - 4) * 16, 16)] = vx * av
                    v1 = tb[e, pl.ds(vo + 16, 16)] * eb[e, pl.ds(vo + 16, 16)]
                    v2 = tb[e, pl.ds(vo + 32, 16)] * eb[e, pl.ds(vo + 32, 16)]
                    hf = h // 4
                    for i in range(3):
                        vv = tb[e, pl.ds(4 * D + i * D + h * 16, 16)]
                        sp[1 + 2 * i + hf, e, pl.ds((h % 4) * 16, 16)] = (
                            vv * v1 + v2 * dvs[i])
                return 0
            lax.fori_loop(0, B, edge, 0)
            pltpu.sync_copy(xlo, acc.at[did], add=True)
            for gg in range(7):
                pltpu.sync_copy(sp.at[gg], msg_hbm.at[gg, pl.ds(base, B), :])
        return 0
    lax.fori_loop(0, JMAX, blockA, 0)
    plsc.subcore_barrier()
    pltpu.sync_copy(acc.at[pl.ds(s * ROWS, ROWS)],
                    px_hbm.at[c, 0, pl.ds(s * ROWS, ROWS), :])

    for p in range(7):
        _zero_acc()
        plsc.subcore_barrier()

        def blockS(j, _, p=p):
            g = j * NW + wid

            @pl.when(g < NBLK)
            def _():
                base = g * B
                pltpu.sync_copy(dst_hbm.at[pl.ds(base, B)], did)
                pltpu.sync_copy(msg_hbm.at[p, pl.ds(base, B), :], xlo)
                pltpu.sync_copy(xlo, acc.at[did], add=True)
            return 0
        lax.fori_loop(0, JMAX, blockS, 0)
        plsc.subcore_barrier()
        if p == 0:
            pltpu.sync_copy(acc.at[pl.ds(s * ROWS, ROWS)],
                            px_hbm.at[c, 1, pl.ds(s * ROWS, ROWS), :])
        else:
            i, hf = (p - 1) // 2, (p - 1) % 2
            pltpu.sync_copy(acc.at[pl.ds(s * ROWS, ROWS)],
                            pv_hbm.at[i, c, hf, pl.ds(s * ROWS, ROWS), :])


_sc_kernel = pl.kernel(
    _sc_body,
    out_type=[
        jax.ShapeDtypeStruct((NC, 2, NPAD, CW), jnp.float32),
        jax.ShapeDtypeStruct((3, NC, 2, NPAD, CW), jnp.float32),
        jax.ShapeDtypeStruct((7, E, CW), jnp.float32),
    ],
    mesh=plsc.VectorSubcoreMesh(core_axis_name="c", subcore_axis_name="s"),
    compiler_params=pltpu.CompilerParams(use_tc_tiling_on_sc=False, needs_layout_passes=False),
    scratch_types=[
        pltpu.VMEM((B,), jnp.int32),
        pltpu.VMEM((B,), jnp.int32),
        pltpu.VMEM((B, D), jnp.float32),
        pltpu.VMEM((B, 7 * D), jnp.float32),
        pltpu.VMEM((B, 4 * D), jnp.float32),
        pltpu.VMEM((B, 64), jnp.float32),
        pltpu.VMEM((B, CW), jnp.float32),
        pltpu.VMEM((7, B, CW), jnp.float32),
        pltpu.VMEM((ZR, CW), jnp.float32),
        pltpu.VMEM_SHARED((NPAD, CW), jnp.float32),
        pltpu.SemaphoreType.DMA,
    ],
)


def _final_body(px_ref, pv_ref, vdot_ref, vec3_ref, Wo_ref, bo_ref,
                dx_ref, dvec_ref):
    xagg = jnp.concatenate(
        [px_ref[0, 0] + px_ref[1, 0], px_ref[0, 1] + px_ref[1, 1]], axis=-1)
    o = xagg @ Wo_ref[...] + bo_ref[...]
    o1 = o[:, :D]
    o2 = o[:, D:2 * D]
    o3 = o[:, 2 * D:]
    dx_ref[...] = vdot_ref[...] * o2 + o3
    for i in range(3):
        vagg = jnp.concatenate(
            [pv_ref[i, 0, 0] + pv_ref[i, 1, 0],
             pv_ref[i, 0, 1] + pv_ref[i, 1, 1]], axis=-1)
        dvec_ref[:, i, :] = vec3_ref[:, i, :] * o1 + vagg


def kernel(x, vec, edge_index, r_ij, f_ij, d_ij, ln_w, ln_b, Wq, bq, Wk, bk,
           Wv, bv, Wo, bo, Wvec, bvec, Wdk, bdk, Wdv, bdv):
    nw = lambda shp: pl.BlockSpec(shp, lambda i: (0,) * len(shp))
    qtab, tsrc, vdot, vec3 = pl.pallas_call(
        _node_body,
        grid=(N // _NB,),
        in_specs=[
            pl.BlockSpec((_NB, D), lambda i: (i, 0)),
            pl.BlockSpec((_NB, 3, D), lambda i: (i, 0, 0)),
            nw((D,)), nw((D,)),
            nw((D, D)), nw((D,)),
            nw((D, D)), nw((D,)),
            nw((D, 3 * D)), nw((3 * D,)),
            nw((D, 3 * D)), nw((3 * D,)),
        ],
        out_specs=[
            pl.BlockSpec((_NB, D), lambda i: (i, 0)),
            pl.BlockSpec((_NB, 7 * D), lambda i: (i, 0)),
            pl.BlockSpec((_NB, D), lambda i: (i, 0)),
            pl.BlockSpec((_NB, 3, D), lambda i: (i, 0, 0)),
        ],
        out_shape=[
            jax.ShapeDtypeStruct((N, D), jnp.float32),
            jax.ShapeDtypeStruct((N, 7 * D), jnp.float32),
            jax.ShapeDtypeStruct((N, D), jnp.float32),
            jax.ShapeDtypeStruct((N, 3, D), jnp.float32),
        ],
    )(x, vec, ln_w, ln_b, Wq, bq, Wk, bk, Wv, bv, Wvec, bvec)

    ef, meta = pl.pallas_call(
        _edge_body,
        grid=(E // _EB,),
        in_specs=[
            pl.BlockSpec((_EB, R), lambda i: (i, 0)),
            pl.BlockSpec((_EB, 1), lambda i: (i, 0)),
            pl.BlockSpec((_EB, 3), lambda i: (i, 0)),
            nw((R, D)), nw((D,)),
            nw((R, 3 * D)), nw((3 * D,)),
        ],
        out_specs=[
            pl.BlockSpec((_EB, 4 * D), lambda i: (i, 0)),
            pl.BlockSpec((_EB, 64), lambda i: (i, 0)),
        ],
        out_shape=[
            jax.ShapeDtypeStruct((E, 4 * D), jnp.float32),
            jax.ShapeDtypeStruct((E, 64), jnp.float32),
        ],
    )(f_ij, r_ij.reshape(E, 1), d_ij, Wdk, bdk, Wdv, bdv)

    src = edge_index[0]
    dst = edge_index[1]
    px, pv, _msg = _sc_kernel(src, dst, qtab, tsrc, ef, meta)

    dx, dvec = pl.pallas_call(
        _final_body,
        grid=(N // _NB,),
        in_specs=[
            pl.BlockSpec((NC, 2, _NB, CW), lambda i: (0, 0, i, 0)),
            pl.BlockSpec((3, NC, 2, _NB, CW), lambda i: (0, 0, 0, i, 0)),
            pl.BlockSpec((_NB, D), lambda i: (i, 0)),
            pl.BlockSpec((_NB, 3, D), lambda i: (i, 0, 0)),
            nw((D, 3 * D)), nw((3 * D,)),
        ],
        out_specs=[
            pl.BlockSpec((_NB, D), lambda i: (i, 0)),
            pl.BlockSpec((_NB, 3, D), lambda i: (i, 0, 0)),
        ],
        out_shape=[
            jax.ShapeDtypeStruct((N, D), jnp.float32),
            jax.ShapeDtypeStruct((N, 3, D), jnp.float32),
        ],
    )(px, pv, vdot, vec3, Wo, bo)
    return (dx, dvec)


# speedup vs baseline: 12.3816x; 1.8605x over previous
import jax
import jax.numpy as jnp
from jax import lax
from jax.experimental import pallas as pl
from jax.experimental.pallas import tpu as pltpu
from jax.experimental.pallas import tpu_sc as plsc

N = 10000
E = 320000
D = 128
H = 8
DH = D // H
R = 50
CUT_UP = 5.0
EPS = 1e-5

NC = 2
NS = 16
NW = NC * NS
EPT = E // NW

BA = 40
JA = EPT // BA
BB = 80
JB = EPT // BB

NPAD = 10240
ROWS = NPAD // NS
ZR = 32

_NB = 1000
_EB = 4000

_SC_PARAMS = pltpu.CompilerParams(use_tc_tiling_on_sc=False,
                                  needs_layout_passes=False)


def _silu(x):
    return x * (1.0 / (1.0 + jnp.exp(-x)))


def _node_body(x_ref, vec_ref, ln_w_ref, ln_b_ref, Wq_ref, bq_ref, Wk_ref,
               bk_ref, Wv_ref, bv_ref, Wvec_ref, bvec_ref,
               q_ref, t_ref, vdot_ref, vec3_ref):
    x = x_ref[...]
    mu = jnp.mean(x, axis=-1, keepdims=True)
    var = jnp.mean((x - mu) ** 2, axis=-1, keepdims=True)
    xn = (x - mu) / jnp.sqrt(var + EPS) * ln_w_ref[...] + ln_b_ref[...]
    q_ref[...] = xn @ Wq_ref[...] + bq_ref[...]
    t_ref[:, 0:D] = xn @ Wk_ref[...] + bk_ref[...]
    t_ref[:, D:4 * D] = xn @ Wv_ref[...] + bv_ref[...]
    Wvec = Wvec_ref[...]
    bvec = bvec_ref[...]
    vdot = jnp.zeros((_NB, D), jnp.float32)
    for i in range(3):
        vi = vec_ref[:, i, :]
        t_ref[:, 4 * D + i * D:4 * D + (i + 1) * D] = vi
        vp = vi @ Wvec + bvec
        vdot = vdot + vp[:, :D] * vp[:, D:2 * D]
        vec3_ref[:, i, :] = vp[:, 2 * D:]
    vdot_ref[...] = vdot


def _edge_body(f_ref, r_ref, d_ref, Wdk_ref, bdk_ref, Wdv_ref, bdv_ref,
               ef_ref, mt_ref):
    f = f_ref[...]
    ef_ref[:, 0:D] = _silu(f @ Wdk_ref[...] + bdk_ref[...])
    ef_ref[:, D:4 * D] = _silu(f @ Wdv_ref[...] + bdv_ref[...])
    r = r_ref[...]
    cut = 0.5 * (jnp.cos(r * jnp.pi / CUT_UP) + 1.0) * (r < CUT_UP).astype(r.dtype)
    col = lax.broadcasted_iota(jnp.int32, (_EB, 64), 1) // 16
    d = d_ref[...]
    m = cut * (col == 0).astype(jnp.float32)
    for i in range(3):
        m = m + d[:, i:i + 1] * (col == i + 1).astype(jnp.float32)
    mt_ref[...] = m


def _scA_body(src_hbm, dst_hbm, q_hbm, t_hbm, ef_hbm, mt_hbm,
              msg_hbm,
              sid, did, qb, tb, eb, mb, sp, semI, semG, semS):
    c = lax.axis_index("c")
    s = lax.axis_index("s")
    wid = s * NC + c
    tbase = wid * EPT

    pltpu.async_copy(src_hbm.at[pl.ds(tbase, BA)], sid, semI)
    pltpu.async_copy(dst_hbm.at[pl.ds(tbase, BA)], did, semI)

    def blockA(j, _):
        base = tbase + j * BA
        pltpu.make_async_copy(src_hbm.at[pl.ds(base, BA)], sid, semI).wait()
        pltpu.make_async_copy(dst_hbm.at[pl.ds(base, BA)], did, semI).wait()
        hg = [pltpu.async_copy(t_hbm.at[sid], tb, semG),
              pltpu.async_copy(q_hbm.at[did], qb, semG),
              pltpu.async_copy(ef_hbm.at[pl.ds(base, BA), :], eb, semG),
              pltpu.async_copy(mt_hbm.at[pl.ds(base, BA), :], mb, semG)]
        for h in hg:
            h.wait()

        @pl.when(j + 1 < JA)
        def _():
            nbase = tbase + (j + 1) * BA
            pltpu.async_copy(src_hbm.at[pl.ds(nbase, BA)], sid, semI)
            pltpu.async_copy(dst_hbm.at[pl.ds(nbase, BA)], did, semI)

        @pl.when(j > 0)
        def _():
            for g in range(4):
                pltpu.make_async_copy(
                    msg_hbm.at[g, pl.ds(base, BA), :], sp.at[g], semS).wait()

        def edge(e, _):
            cutv = mb[e, pl.ds(0, 16)]
            dvs = tuple(mb[e, pl.ds(16 * (i + 1), 16)] for i in range(3))
            for h in range(H):
                t = (qb[e, pl.ds(h * 16, 16)]
                     * tb[e, pl.ds(h * 16, 16)]
                     * eb[e, pl.ds(h * 16, 16)])
                sv = jnp.broadcast_to(jnp.sum(t), (16,))
                av = sv * (1.0 / (1.0 + jnp.exp(-sv))) * cutv
                vo = D + h * 48
                vx = tb[e, pl.ds(vo, 16)] * eb[e, pl.ds(vo, 16)]
                sp[0, e, pl.ds(h * 16, 16)] = vx * av
                v1 = tb[e, pl.ds(vo + 16, 16)] * eb[e, pl.ds(vo + 16, 16)]
                v2 = tb[e, pl.ds(vo + 32, 16)] * eb[e, pl.ds(vo + 32, 16)]
                for i in range(3):
                    vv = tb[e, pl.ds(4 * D + i * D + h * 16, 16)]
                    sp[1 + i, e, pl.ds(h * 16, 16)] = vv * v1 + v2 * dvs[i]
            return 0
        lax.fori_loop(0, BA, edge, 0)

        for g in range(4):
            pltpu.async_copy(sp.at[g], msg_hbm.at[g, pl.ds(base, BA), :], semS)
        return 0
    lax.fori_loop(0, JA, blockA, 0)
    for g in range(4):
        pltpu.make_async_copy(
            msg_hbm.at[g, pl.ds(tbase, BA), :], sp.at[g], semS).wait()


_scA_kernel = pl.kernel(
    _scA_body,
    out_type=[jax.ShapeDtypeStruct((4, E, D), jnp.float32)],
    mesh=plsc.VectorSubcoreMesh(core_axis_name="c", subcore_axis_name="s"),
    compiler_params=_SC_PARAMS,
    scratch_types=[
        pltpu.VMEM((BA,), jnp.int32),
        pltpu.VMEM((BA,), jnp.int32),
        pltpu.VMEM((BA, D), jnp.float32),
        pltpu.VMEM((BA, 7 * D), jnp.float32),
        pltpu.VMEM((BA, 4 * D), jnp.float32),
        pltpu.VMEM((BA, 64), jnp.float32),
        pltpu.VMEM((4, BA, D), jnp.float32),
        pltpu.SemaphoreType.DMA,
        pltpu.SemaphoreType.DMA,
        pltpu.SemaphoreType.DMA,
    ],
)


def _scB_body(dst_hbm, msg_hbm, px_hbm, pv_hbm,
              did, rbuf, zb, acc, sem):
    c = lax.axis_index("c")
    s = lax.axis_index("s")
    wid = s * NC + c
    tbase = wid * EPT

    def _zrow(i, _):
        for l in range(D // 16):
            zb[i, pl.ds(l * 16, 16)] = jnp.zeros((16,), jnp.float32)
        return 0
    lax.fori_loop(0, ZR, _zrow, 0)

    def _zero_acc():
        for rep in range(ROWS // ZR):
            pltpu.sync_copy(zb, acc.at[pl.ds(s * ROWS + rep * ZR, ZR)])

    for p in range(4):
        _zero_acc()
        plsc.subcore_barrier()

        def blockB(j, _, p=p):
            base = tbase + j * BB
            h1 = pltpu.async_copy(dst_hbm.at[pl.ds(base, BB)], did, sem)
            h2 = pltpu.async_copy(msg_hbm.at[p, pl.ds(base, BB), :], rbuf, sem)
            h1.wait()
            h2.wait()
            pltpu.sync_copy(rbuf, acc.at[did], add=True)
            return 0
        lax.fori_loop(0, JB, blockB, 0)
        plsc.subcore_barrier()
        if p == 0:
            pltpu.sync_copy(acc.at[pl.ds(s * ROWS, ROWS)],
                            px_hbm.at[c, pl.ds(s * ROWS, ROWS), :])
        else:
            pltpu.sync_copy(acc.at[pl.ds(s * ROWS, ROWS)],
                            pv_hbm.at[p - 1, c, pl.ds(s * ROWS, ROWS), :])
        plsc.subcore_barrier()


_scB_kernel = pl.kernel(
    _scB_body,
    out_type=[
        jax.ShapeDtypeStruct((NC, NPAD, D), jnp.float32),
        jax.ShapeDtypeStruct((3, NC, NPAD, D), jnp.float32),
    ],
    mesh=plsc.VectorSubcoreMesh(core_axis_name="c", subcore_axis_name="s"),
    compiler_params=_SC_PARAMS,
    scratch_types=[
        pltpu.VMEM((BB,), jnp.int32),
        pltpu.VMEM((BB, D), jnp.float32),
        pltpu.VMEM((ZR, D), jnp.float32),
        pltpu.VMEM_SHARED((NPAD, D), jnp.float32),
        pltpu.SemaphoreType.DMA,
    ],
)


def _final_body(px_ref, pv_ref, vdot_ref, vec3_ref, Wo_ref, bo_ref,
                dx_ref, dvec_ref):
    xagg = px_ref[0] + px_ref[1]
    o = xagg @ Wo_ref[...] + bo_ref[...]
    o1 = o[:, :D]
    o2 = o[:, D:2 * D]
    o3 = o[:, 2 * D:]
    dx_ref[...] = vdot_ref[...] * o2 + o3
    for i in range(3):
        dvec_ref[:, i, :] = (vec3_ref[:, i, :] * o1
                             + pv_ref[i, 0] + pv_ref[i, 1])


def kernel(x, vec, edge_index, r_ij, f_ij, d_ij, ln_w, ln_b, Wq, bq, Wk, bk,
           Wv, bv, Wo, bo, Wvec, bvec, Wdk, bdk, Wdv, bdv):
    nw = lambda shp: pl.BlockSpec(shp, lambda i: (0,) * len(shp))
    qtab, tsrc, vdot, vec3 = pl.pallas_call(
        _node_body,
        grid=(N // _NB,),
        in_specs=[
            pl.BlockSpec((_NB, D), lambda i: (i, 0)),
            pl.BlockSpec((_NB, 3, D), lambda i: (i, 0, 0)),
            nw((D,)), nw((D,)),
            nw((D, D)), nw((D,)),
            nw((D, D)), nw((D,)),
            nw((D, 3 * D)), nw((3 * D,)),
            nw((D, 3 * D)), nw((3 * D,)),
        ],
        out_specs=[
            pl.BlockSpec((_NB, D), lambda i: (i, 0)),
            pl.BlockSpec((_NB, 7 * D), lambda i: (i, 0)),
            pl.BlockSpec((_NB, D), lambda i: (i, 0)),
            pl.BlockSpec((_NB, 3, D), lambda i: (i, 0, 0)),
        ],
        out_shape=[
            jax.ShapeDtypeStruct((N, D), jnp.float32),
            jax.ShapeDtypeStruct((N, 7 * D), jnp.float32),
            jax.ShapeDtypeStruct((N, D), jnp.float32),
            jax.ShapeDtypeStruct((N, 3, D), jnp.float32),
        ],
    )(x, vec, ln_w, ln_b, Wq, bq, Wk, bk, Wv, bv, Wvec, bvec)

    ef, meta = pl.pallas_call(
        _edge_body,
        grid=(E // _EB,),
        in_specs=[
            pl.BlockSpec((_EB, R), lambda i: (i, 0)),
            pl.BlockSpec((_EB, 1), lambda i: (i, 0)),
            pl.BlockSpec((_EB, 3), lambda i: (i, 0)),
            nw((R, D)), nw((D,)),
            nw((R, 3 * D)), nw((3 * D,)),
        ],
        out_specs=[
            pl.BlockSpec((_EB, 4 * D), lambda i: (i, 0)),
            pl.BlockSpec((_EB, 64), lambda i: (i, 0)),
        ],
        out_shape=[
            jax.ShapeDtypeStruct((E, 4 * D), jnp.float32),
            jax.ShapeDtypeStruct((E, 64), jnp.float32),
        ],
    )(f_ij, r_ij.reshape(E, 1), d_ij, Wdk, bdk, Wdv, bdv)

    src = edge_index[0]
    dst = edge_index[1]
    (msg,) = _scA_kernel(src, dst, qtab, tsrc, ef, meta)
    px, pv = _scB_kernel(dst, msg)

    dx, dvec = pl.pallas_call(
        _final_body,
        grid=(N // _NB,),
        in_specs=[
            pl.BlockSpec((NC, _NB, D), lambda i: (0, i, 0)),
            pl.BlockSpec((3, NC, _NB, D), lambda i: (0, 0, i, 0)),
            pl.BlockSpec((_NB, D), lambda i: (i, 0)),
            pl.BlockSpec((_NB, 3, D), lambda i: (i, 0, 0)),
            nw((D, 3 * D)), nw((3 * D,)),
        ],
        out_specs=[
            pl.BlockSpec((_NB, D), lambda i: (i, 0)),
            pl.BlockSpec((_NB, 3, D), lambda i: (i, 0, 0)),
        ],
        out_shape=[
            jax.ShapeDtypeStruct((N, D), jnp.float32),
            jax.ShapeDtypeStruct((N, 3, D), jnp.float32),
        ],
    )(px, pv, vdot, vec3, Wo, bo)
    return (dx, dvec)


# speedup vs baseline: 12.7422x; 1.0291x over previous
import jax
import jax.numpy as jnp
from jax import lax
from jax.experimental import pallas as pl
from jax.experimental.pallas import tpu as pltpu
from jax.experimental.pallas import tpu_sc as plsc

N = 10000
E = 320000
D = 128
H = 8
DH = D // H
R = 50
CUT_UP = 5.0
EPS = 1e-5

NC = 2
NS = 16
NW = NC * NS
EPT = E // NW

BA = 40
JA = EPT // BA
BB = 80
JB = EPT // BB

NPAD = 10240
ROWS = NPAD // NS
ZR = 32

_NB = 1000
_EB = 4000

_SC_PARAMS = pltpu.CompilerParams(use_tc_tiling_on_sc=False,
                                  needs_layout_passes=False)


def _silu(x):
    return x * (1.0 / (1.0 + jnp.exp(-x)))


def _node_body(x_ref, vec_ref, ln_w_ref, ln_b_ref, Wq_ref, bq_ref, Wk_ref,
               bk_ref, Wv_ref, bv_ref, Wvec_ref, bvec_ref,
               q_ref, t_ref, vdot_ref, vec3_ref):
    x = x_ref[...]
    mu = jnp.mean(x, axis=-1, keepdims=True)
    var = jnp.mean((x - mu) ** 2, axis=-1, keepdims=True)
    xn = (x - mu) / jnp.sqrt(var + EPS) * ln_w_ref[...] + ln_b_ref[...]
    q_ref[...] = xn @ Wq_ref[...] + bq_ref[...]
    t_ref[:, 0:D] = xn @ Wk_ref[...] + bk_ref[...]
    t_ref[:, D:4 * D] = xn @ Wv_ref[...] + bv_ref[...]
    Wvec = Wvec_ref[...]
    bvec = bvec_ref[...]
    vdot = jnp.zeros((_NB, D), jnp.float32)
    for i in range(3):
        vi = vec_ref[:, i, :]
        t_ref[:, 4 * D + i * D:4 * D + (i + 1) * D] = vi
        vp = vi @ Wvec + bvec
        vdot = vdot + vp[:, :D] * vp[:, D:2 * D]
        vec3_ref[:, i, :] = vp[:, 2 * D:]
    vdot_ref[...] = vdot


def _edge_body(f_ref, r_ref, d_ref, Wdk_ref, bdk_ref, Wdv_ref, bdv_ref,
               ef_ref, mt_ref):
    f = f_ref[...]
    ef_ref[:, 0:D] = _silu(f @ Wdk_ref[...] + bdk_ref[...])
    ef_ref[:, D:4 * D] = _silu(f @ Wdv_ref[...] + bdv_ref[...])
    r = r_ref[...]
    cut = 0.5 * (jnp.cos(r * jnp.pi / CUT_UP) + 1.0) * (r < CUT_UP).astype(r.dtype)
    col = lax.broadcasted_iota(jnp.int32, (_EB, 64), 1) // 16
    d = d_ref[...]
    m = cut * (col == 0).astype(jnp.float32)
    for i in range(3):
        m = m + d[:, i:i + 1] * (col == i + 1).astype(jnp.float32)
    mt_ref[...] = m


def _scA_body(ei_hbm, q_hbm, t_hbm, ef_hbm, mt_hbm,
              msg_hbm,
              sid, did, qb, tb, eb, mb, sp, semI, semG, semS):
    c = lax.axis_index("c")
    s = lax.axis_index("s")
    wid = s * NC + c
    tbase = wid * EPT

    src_hbm = ei_hbm.at[0]
    dst_hbm = ei_hbm.at[1]
    pltpu.async_copy(src_hbm.at[pl.ds(tbase, BA)], sid, semI)
    pltpu.async_copy(dst_hbm.at[pl.ds(tbase, BA)], did, semI)

    def blockA(j, _):
        base = tbase + j * BA
        pltpu.make_async_copy(src_hbm.at[pl.ds(base, BA)], sid, semI).wait()
        pltpu.make_async_copy(dst_hbm.at[pl.ds(base, BA)], did, semI).wait()
        hg = [pltpu.async_copy(t_hbm.at[sid], tb, semG),
              pltpu.async_copy(q_hbm.at[did], qb, semG),
              pltpu.async_copy(ef_hbm.at[pl.ds(base, BA), :], eb, semG),
              pltpu.async_copy(mt_hbm.at[pl.ds(base, BA), :], mb, semG)]
        for h in hg:
            h.wait()

        @pl.when(j + 1 < JA)
        def _():
            nbase = tbase + (j + 1) * BA
            pltpu.async_copy(src_hbm.at[pl.ds(nbase, BA)], sid, semI)
            pltpu.async_copy(dst_hbm.at[pl.ds(nbase, BA)], did, semI)

        @pl.when(j > 0)
        def _():
            for g in range(4):
                pltpu.make_async_copy(
                    msg_hbm.at[g, pl.ds(base, BA), :], sp.at[g], semS).wait()

        @plsc.parallel_loop(0, BA, unroll=2)
        def edge(e):
            cutv = mb[e, pl.ds(0, 16)]
            dvs = tuple(mb[e, pl.ds(16 * (i + 1), 16)] for i in range(3))
            for h in range(H):
                t = (qb[e, pl.ds(h * 16, 16)]
                     * tb[e, pl.ds(h * 16, 16)]
                     * eb[e, pl.ds(h * 16, 16)])
                sv = jnp.broadcast_to(jnp.sum(t), (16,))
                av = sv * (1.0 / (1.0 + jnp.exp(-sv))) * cutv
                vo = D + h * 48
                vx = tb[e, pl.ds(vo, 16)] * eb[e, pl.ds(vo, 16)]
                sp[0, e, pl.ds(h * 16, 16)] = vx * av
                v1 = tb[e, pl.ds(vo + 16, 16)] * eb[e, pl.ds(vo + 16, 16)]
                v2 = tb[e, pl.ds(vo + 32, 16)] * eb[e, pl.ds(vo + 32, 16)]
                for i in range(3):
                    vv = tb[e, pl.ds(4 * D + i * D + h * 16, 16)]
                    sp[1 + i, e, pl.ds(h * 16, 16)] = vv * v1 + v2 * dvs[i]

        for g in range(4):
            pltpu.async_copy(sp.at[g], msg_hbm.at[g, pl.ds(base, BA), :], semS)
        return 0
    lax.fori_loop(0, JA, blockA, 0)
    for g in range(4):
        pltpu.make_async_copy(
            msg_hbm.at[g, pl.ds(tbase, BA), :], sp.at[g], semS).wait()


_scA_kernel = pl.kernel(
    _scA_body,
    out_type=[jax.ShapeDtypeStruct((4, E, D), jnp.float32)],
    mesh=plsc.VectorSubcoreMesh(core_axis_name="c", subcore_axis_name="s"),
    compiler_params=_SC_PARAMS,
    scratch_types=[
        pltpu.VMEM((BA,), jnp.int32),
        pltpu.VMEM((BA,), jnp.int32),
        pltpu.VMEM((BA, D), jnp.float32),
        pltpu.VMEM((BA, 7 * D), jnp.float32),
        pltpu.VMEM((BA, 4 * D), jnp.float32),
        pltpu.VMEM((BA, 64), jnp.float32),
        pltpu.VMEM((4, BA, D), jnp.float32),
        pltpu.SemaphoreType.DMA,
        pltpu.SemaphoreType.DMA,
        pltpu.SemaphoreType.DMA,
    ],
)


def _scB_body(ei_hbm, msg_hbm, px_hbm, pv_hbm,
              did, rbuf, zb, acc, sem):
    dst_hbm = ei_hbm.at[1]
    c = lax.axis_index("c")
    s = lax.axis_index("s")
    wid = s * NC + c
    tbase = wid * EPT

    def _zrow(i, _):
        for l in range(D // 16):
            zb[i, pl.ds(l * 16, 16)] = jnp.zeros((16,), jnp.float32)
        return 0
    lax.fori_loop(0, ZR, _zrow, 0)

    def _zero_acc():
        for rep in range(ROWS // ZR):
            pltpu.sync_copy(zb, acc.at[pl.ds(s * ROWS + rep * ZR, ZR)])

    for p in range(4):
        _zero_acc()
        plsc.subcore_barrier()

        def blockB(j, _, p=p):
            base = tbase + j * BB
            h1 = pltpu.async_copy(dst_hbm.at[pl.ds(base, BB)], did, sem)
            h2 = pltpu.async_copy(msg_hbm.at[p, pl.ds(base, BB), :], rbuf, sem)
            h1.wait()
            h2.wait()
            pltpu.sync_copy(rbuf, acc.at[did], add=True)
            return 0
        lax.fori_loop(0, JB, blockB, 0)
        plsc.subcore_barrier()
        if p == 0:
            pltpu.sync_copy(acc.at[pl.ds(s * ROWS, ROWS)],
                            px_hbm.at[c, pl.ds(s * ROWS, ROWS), :])
        else:
            pltpu.sync_copy(acc.at[pl.ds(s * ROWS, ROWS)],
                            pv_hbm.at[p - 1, c, pl.ds(s * ROWS, ROWS), :])
        plsc.subcore_barrier()


_scB_kernel = pl.kernel(
    _scB_body,
    out_type=[
        jax.ShapeDtypeStruct((NC, NPAD, D), jnp.float32),
        jax.ShapeDtypeStruct((3, NC, NPAD, D), jnp.float32),
    ],
    mesh=plsc.VectorSubcoreMesh(core_axis_name="c", subcore_axis_name="s"),
    compiler_params=_SC_PARAMS,
    scratch_types=[
        pltpu.VMEM((BB,), jnp.int32),
        pltpu.VMEM((BB, D), jnp.float32),
        pltpu.VMEM((ZR, D), jnp.float32),
        pltpu.VMEM_SHARED((NPAD, D), jnp.float32),
        pltpu.SemaphoreType.DMA,
    ],
)


def _final_body(px_ref, pv_ref, vdot_ref, vec3_ref, Wo_ref, bo_ref,
                dx_ref, dvec_ref):
    xagg = px_ref[0] + px_ref[1]
    o = xagg @ Wo_ref[...] + bo_ref[...]
    o1 = o[:, :D]
    o2 = o[:, D:2 * D]
    o3 = o[:, 2 * D:]
    dx_ref[...] = vdot_ref[...] * o2 + o3
    for i in range(3):
        dvec_ref[:, i, :] = (vec3_ref[:, i, :] * o1
                             + pv_ref[i, 0] + pv_ref[i, 1])


def kernel(x, vec, edge_index, r_ij, f_ij, d_ij, ln_w, ln_b, Wq, bq, Wk, bk,
           Wv, bv, Wo, bo, Wvec, bvec, Wdk, bdk, Wdv, bdv):
    nw = lambda shp: pl.BlockSpec(shp, lambda i: (0,) * len(shp))
    qtab, tsrc, vdot, vec3 = pl.pallas_call(
        _node_body,
        grid=(N // _NB,),
        in_specs=[
            pl.BlockSpec((_NB, D), lambda i: (i, 0)),
            pl.BlockSpec((_NB, 3, D), lambda i: (i, 0, 0)),
            nw((D,)), nw((D,)),
            nw((D, D)), nw((D,)),
            nw((D, D)), nw((D,)),
            nw((D, 3 * D)), nw((3 * D,)),
            nw((D, 3 * D)), nw((3 * D,)),
        ],
        out_specs=[
            pl.BlockSpec((_NB, D), lambda i: (i, 0)),
            pl.BlockSpec((_NB, 7 * D), lambda i: (i, 0)),
            pl.BlockSpec((_NB, D), lambda i: (i, 0)),
            pl.BlockSpec((_NB, 3, D), lambda i: (i, 0, 0)),
        ],
        out_shape=[
            jax.ShapeDtypeStruct((N, D), jnp.float32),
            jax.ShapeDtypeStruct((N, 7 * D), jnp.float32),
            jax.ShapeDtypeStruct((N, D), jnp.float32),
            jax.ShapeDtypeStruct((N, 3, D), jnp.float32),
        ],
    )(x, vec, ln_w, ln_b, Wq, bq, Wk, bk, Wv, bv, Wvec, bvec)

    ef, meta = pl.pallas_call(
        _edge_body,
        grid=(E // _EB,),
        in_specs=[
            pl.BlockSpec((_EB, R), lambda i: (i, 0)),
            pl.BlockSpec((_EB, 1), lambda i: (i, 0)),
            pl.BlockSpec((_EB, 3), lambda i: (i, 0)),
            nw((R, D)), nw((D,)),
            nw((R, 3 * D)), nw((3 * D,)),
        ],
        out_specs=[
            pl.BlockSpec((_EB, 4 * D), lambda i: (i, 0)),
            pl.BlockSpec((_EB, 64), lambda i: (i, 0)),
        ],
        out_shape=[
            jax.ShapeDtypeStruct((E, 4 * D), jnp.float32),
            jax.ShapeDtypeStruct((E, 64), jnp.float32),
        ],
    )(f_ij, r_ij.reshape(E, 1), d_ij, Wdk, bdk, Wdv, bdv)

    (msg,) = _scA_kernel(edge_index, qtab, tsrc, ef, meta)
    px, pv = _scB_kernel(edge_index, msg)

    dx, dvec = pl.pallas_call(
        _final_body,
        grid=(N // _NB,),
        in_specs=[
            pl.BlockSpec((NC, _NB, D), lambda i: (0, i, 0)),
            pl.BlockSpec((3, NC, _NB, D), lambda i: (0, 0, i, 0)),
            pl.BlockSpec((_NB, D), lambda i: (i, 0)),
            pl.BlockSpec((_NB, 3, D), lambda i: (i, 0, 0)),
            nw((D, 3 * D)), nw((3 * D,)),
        ],
        out_specs=[
            pl.BlockSpec((_NB, D), lambda i: (i, 0)),
            pl.BlockSpec((_NB, 3, D), lambda i: (i, 0, 0)),
        ],
        out_shape=[
            jax.ShapeDtypeStruct((N, D), jnp.float32),
            jax.ShapeDtypeStruct((N, 3, D), jnp.float32),
        ],
    )(px, pv, vdot, vec3, Wo, bo)
    return (dx, dvec)


# speedup vs baseline: 12.9346x; 1.0151x over previous
import jax
import jax.numpy as jnp
from jax import lax
from jax.experimental import pallas as pl
from jax.experimental.pallas import tpu as pltpu
from jax.experimental.pallas import tpu_sc as plsc

N = 10000
E = 320000
D = 128
H = 8
DH = D // H
R = 50
CUT_UP = 5.0
EPS = 1e-5

NC = 2
NS = 16
NW = NC * NS
EPT = E // NW

BA = 40
JA = EPT // BA
BB = 80
JB = EPT // BB

NPAD = 10240
ROWS = NPAD // NS
ZR = 32

_NB = 1000
_EB = 4000

_SC_PARAMS = pltpu.CompilerParams(use_tc_tiling_on_sc=False,
                                  needs_layout_passes=False)


def _silu(x):
    return x * (1.0 / (1.0 + jnp.exp(-x)))


def _node_body(x_ref, vec_ref, ln_w_ref, ln_b_ref, Wq_ref, bq_ref, Wk_ref,
               bk_ref, Wv_ref, bv_ref, Wvec_ref, bvec_ref,
               q_ref, t_ref, vdot_ref, vec3_ref):
    x = x_ref[...]
    mu = jnp.mean(x, axis=-1, keepdims=True)
    var = jnp.mean((x - mu) ** 2, axis=-1, keepdims=True)
    xn = (x - mu) / jnp.sqrt(var + EPS) * ln_w_ref[...] + ln_b_ref[...]
    q_ref[...] = xn @ Wq_ref[...] + bq_ref[...]
    t_ref[:, 0:D] = xn @ Wk_ref[...] + bk_ref[...]
    t_ref[:, D:4 * D] = xn @ Wv_ref[...] + bv_ref[...]
    Wvec = Wvec_ref[...]
    bvec = bvec_ref[...]
    vdot = jnp.zeros((_NB, D), jnp.float32)
    for i in range(3):
        vi = vec_ref[:, i, :]
        t_ref[:, 4 * D + i * D:4 * D + (i + 1) * D] = vi
        vp = vi @ Wvec + bvec
        vdot = vdot + vp[:, :D] * vp[:, D:2 * D]
        vec3_ref[:, i, :] = vp[:, 2 * D:]
    vdot_ref[...] = vdot


def _edge_body(f_ref, r_ref, d_ref, Wdk_ref, bdk_ref, Wdv_ref, bdv_ref,
               ef_ref, mt_ref):
    f = f_ref[...]
    ef_ref[:, 0:D] = _silu(f @ Wdk_ref[...] + bdk_ref[...])
    ef_ref[:, D:4 * D] = _silu(f @ Wdv_ref[...] + bdv_ref[...])
    r = r_ref[...]
    cut = 0.5 * (jnp.cos(r * jnp.pi / CUT_UP) + 1.0) * (r < CUT_UP).astype(r.dtype)
    col = lax.broadcasted_iota(jnp.int32, (_EB, 64), 1) // 16
    d = d_ref[...]
    m = cut * (col == 0).astype(jnp.float32)
    for i in range(3):
        m = m + d[:, i:i + 1] * (col == i + 1).astype(jnp.float32)
    mt_ref[...] = m


def _scA_body(ei_hbm, q_hbm, t_hbm, ef_hbm, mt_hbm,
              msg_hbm,
              sid, did, qb, tb, eb, mb, sp, semI, semG, semS):
    c = lax.axis_index("c")
    s = lax.axis_index("s")
    wid = s * NC + c
    tbase = wid * EPT

    src_hbm = ei_hbm.at[0]
    dst_hbm = ei_hbm.at[1]
    pltpu.async_copy(src_hbm.at[pl.ds(tbase, BA)], sid, semI)
    pltpu.async_copy(dst_hbm.at[pl.ds(tbase, BA)], did, semI)

    def blockA(j, _):
        base = tbase + j * BA
        pltpu.make_async_copy(src_hbm.at[pl.ds(base, BA)], sid, semI).wait()
        pltpu.make_async_copy(dst_hbm.at[pl.ds(base, BA)], did, semI).wait()
        hg = [pltpu.async_copy(t_hbm.at[sid], tb, semG),
              pltpu.async_copy(q_hbm.at[did], qb, semG),
              pltpu.async_copy(ef_hbm.at[pl.ds(base, BA), :], eb, semG),
              pltpu.async_copy(mt_hbm.at[pl.ds(base, BA), :], mb, semG)]
        for h in hg:
            h.wait()

        @pl.when(j + 1 < JA)
        def _():
            nbase = tbase + (j + 1) * BA
            pltpu.async_copy(src_hbm.at[pl.ds(nbase, BA)], sid, semI)
            pltpu.async_copy(dst_hbm.at[pl.ds(nbase, BA)], did, semI)

        @pl.when(j > 0)
        def _():
            for g in range(4):
                pltpu.make_async_copy(
                    msg_hbm.at[g, pl.ds(base, BA), :], sp.at[g], semS).wait()

        @plsc.parallel_loop(0, BA, unroll=4)
        def edge(e):
            cutv = mb[e, pl.ds(0, 16)]
            dvs = tuple(mb[e, pl.ds(16 * (i + 1), 16)] for i in range(3))
            for h in range(H):
                t = (qb[e, pl.ds(h * 16, 16)]
                     * tb[e, pl.ds(h * 16, 16)]
                     * eb[e, pl.ds(h * 16, 16)])
                sv = jnp.broadcast_to(jnp.sum(t), (16,))
                av = sv * (1.0 / (1.0 + jnp.exp(-sv))) * cutv
                vo = D + h * 48
                vx = tb[e, pl.ds(vo, 16)] * eb[e, pl.ds(vo, 16)]
                sp[0, e, pl.ds(h * 16, 16)] = vx * av
                v1 = tb[e, pl.ds(vo + 16, 16)] * eb[e, pl.ds(vo + 16, 16)]
                v2 = tb[e, pl.ds(vo + 32, 16)] * eb[e, pl.ds(vo + 32, 16)]
                for i in range(3):
                    vv = tb[e, pl.ds(4 * D + i * D + h * 16, 16)]
                    sp[1 + i, e, pl.ds(h * 16, 16)] = vv * v1 + v2 * dvs[i]

        for g in range(4):
            pltpu.async_copy(sp.at[g], msg_hbm.at[g, pl.ds(base, BA), :], semS)
        return 0
    lax.fori_loop(0, JA, blockA, 0)
    for g in range(4):
        pltpu.make_async_copy(
            msg_hbm.at[g, pl.ds(tbase, BA), :], sp.at[g], semS).wait()


_scA_kernel = pl.kernel(
    _scA_body,
    out_type=[jax.ShapeDtypeStruct((4, E, D), jnp.float32)],
    mesh=plsc.VectorSubcoreMesh(core_axis_name="c", subcore_axis_name="s"),
    compiler_params=_SC_PARAMS,
    scratch_types=[
        pltpu.VMEM((BA,), jnp.int32),
        pltpu.VMEM((BA,), jnp.int32),
        pltpu.VMEM((BA, D), jnp.float32),
        pltpu.VMEM((BA, 7 * D), jnp.float32),
        pltpu.VMEM((BA, 4 * D), jnp.float32),
        pltpu.VMEM((BA, 64), jnp.float32),
        pltpu.VMEM((4, BA, D), jnp.float32),
        pltpu.SemaphoreType.DMA,
        pltpu.SemaphoreType.DMA,
        pltpu.SemaphoreType.DMA,
    ],
)


def _scB_body(ei_hbm, msg_hbm, px_hbm, pv_hbm,
              did, rbuf, zb, acc, sem):
    dst_hbm = ei_hbm.at[1]
    c = lax.axis_index("c")
    s = lax.axis_index("s")
    wid = s * NC + c
    tbase = wid * EPT

    def _zrow(i, _):
        for l in range(D // 16):
            zb[i, pl.ds(l * 16, 16)] = jnp.zeros((16,), jnp.float32)
        return 0
    lax.fori_loop(0, ZR, _zrow, 0)

    def _zero_acc():
        for rep in range(ROWS // ZR):
            pltpu.sync_copy(zb, acc.at[pl.ds(s * ROWS + rep * ZR, ZR)])

    for p in range(4):
        _zero_acc()
        plsc.subcore_barrier()

        def blockB(j, _, p=p):
            base = tbase + j * BB
            h1 = pltpu.async_copy(dst_hbm.at[pl.ds(base, BB)], did, sem)
            h2 = pltpu.async_copy(msg_hbm.at[p, pl.ds(base, BB), :], rbuf, sem)
            h1.wait()
            h2.wait()
            pltpu.sync_copy(rbuf, acc.at[did], add=True)
            return 0
        lax.fori_loop(0, JB, blockB, 0)
        plsc.subcore_barrier()
        if p == 0:
            pltpu.sync_copy(acc.at[pl.ds(s * ROWS, ROWS)],
                            px_hbm.at[c, pl.ds(s * ROWS, ROWS), :])
        else:
            pltpu.sync_copy(acc.at[pl.ds(s * ROWS, ROWS)],
                            pv_hbm.at[p - 1, c, pl.ds(s * ROWS, ROWS), :])
        plsc.subcore_barrier()


_scB_kernel = pl.kernel(
    _scB_body,
    out_type=[
        jax.ShapeDtypeStruct((NC, NPAD, D), jnp.float32),
        jax.ShapeDtypeStruct((3, NC, NPAD, D), jnp.float32),
    ],
    mesh=plsc.VectorSubcoreMesh(core_axis_name="c", subcore_axis_name="s"),
    compiler_params=_SC_PARAMS,
    scratch_types=[
        pltpu.VMEM((BB,), jnp.int32),
        pltpu.VMEM((BB, D), jnp.float32),
        pltpu.VMEM((ZR, D), jnp.float32),
        pltpu.VMEM_SHARED((NPAD, D), jnp.float32),
        pltpu.SemaphoreType.DMA,
    ],
)


def _final_body(px_ref, pv_ref, vdot_ref, vec3_ref, Wo_ref, bo_ref,
                dx_ref, dvec_ref):
    xagg = px_ref[0] + px_ref[1]
    o = xagg @ Wo_ref[...] + bo_ref[...]
    o1 = o[:, :D]
    o2 = o[:, D:2 * D]
    o3 = o[:, 2 * D:]
    dx_ref[...] = vdot_ref[...] * o2 + o3
    for i in range(3):
        dvec_ref[:, i, :] = (vec3_ref[:, i, :] * o1
                             + pv_ref[i, 0] + pv_ref[i, 1])


def kernel(x, vec, edge_index, r_ij, f_ij, d_ij, ln_w, ln_b, Wq, bq, Wk, bk,
           Wv, bv, Wo, bo, Wvec, bvec, Wdk, bdk, Wdv, bdv):
    nw = lambda shp: pl.BlockSpec(shp, lambda i: (0,) * len(shp))
    qtab, tsrc, vdot, vec3 = pl.pallas_call(
        _node_body,
        grid=(N // _NB,),
        in_specs=[
            pl.BlockSpec((_NB, D), lambda i: (i, 0)),
            pl.BlockSpec((_NB, 3, D), lambda i: (i, 0, 0)),
            nw((D,)), nw((D,)),
            nw((D, D)), nw((D,)),
            nw((D, D)), nw((D,)),
            nw((D, 3 * D)), nw((3 * D,)),
            nw((D, 3 * D)), nw((3 * D,)),
        ],
        out_specs=[
            pl.BlockSpec((_NB, D), lambda i: (i, 0)),
            pl.BlockSpec((_NB, 7 * D), lambda i: (i, 0)),
            pl.BlockSpec((_NB, D), lambda i: (i, 0)),
            pl.BlockSpec((_NB, 3, D), lambda i: (i, 0, 0)),
        ],
        out_shape=[
            jax.ShapeDtypeStruct((N, D), jnp.float32),
            jax.ShapeDtypeStruct((N, 7 * D), jnp.float32),
            jax.ShapeDtypeStruct((N, D), jnp.float32),
            jax.ShapeDtypeStruct((N, 3, D), jnp.float32),
        ],
    )(x, vec, ln_w, ln_b, Wq, bq, Wk, bk, Wv, bv, Wvec, bvec)

    ef, meta = pl.pallas_call(
        _edge_body,
        grid=(E // _EB,),
        in_specs=[
            pl.BlockSpec((_EB, R), lambda i: (i, 0)),
            pl.BlockSpec((_EB, 1), lambda i: (i, 0)),
            pl.BlockSpec((_EB, 3), lambda i: (i, 0)),
            nw((R, D)), nw((D,)),
            nw((R, 3 * D)), nw((3 * D,)),
        ],
        out_specs=[
            pl.BlockSpec((_EB, 4 * D), lambda i: (i, 0)),
            pl.BlockSpec((_EB, 64), lambda i: (i, 0)),
        ],
        out_shape=[
            jax.ShapeDtypeStruct((E, 4 * D), jnp.float32),
            jax.ShapeDtypeStruct((E, 64), jnp.float32),
        ],
    )(f_ij, r_ij.reshape(E, 1), d_ij, Wdk, bdk, Wdv, bdv)

    (msg,) = _scA_kernel(edge_index, qtab, tsrc, ef, meta)
    px, pv = _scB_kernel(edge_index, msg)

    dx, dvec = pl.pallas_call(
        _final_body,
        grid=(N // _NB,),
        in_specs=[
            pl.BlockSpec((NC, _NB, D), lambda i: (0, i, 0)),
            pl.BlockSpec((3, NC, _NB, D), lambda i: (0, 0, i, 0)),
            pl.BlockSpec((_NB, D), lambda i: (i, 0)),
            pl.BlockSpec((_NB, 3, D), lambda i: (i, 0, 0)),
            nw((D, 3 * D)), nw((3 * D,)),
        ],
        out_specs=[
            pl.BlockSpec((_NB, D), lambda i: (i, 0)),
            pl.BlockSpec((_NB, 3, D), lambda i: (i, 0, 0)),
        ],
        out_shape=[
            jax.ShapeDtypeStruct((N, D), jnp.float32),
            jax.ShapeDtypeStruct((N, 3, D), jnp.float32),
        ],
    )(px, pv, vdot, vec3, Wo, bo)
    return (dx, dvec)


# speedup vs baseline: 18.6749x; 1.4438x over previous
import jax
import jax.numpy as jnp
from jax import lax
from jax.experimental import pallas as pl
from jax.experimental.pallas import tpu as pltpu
from jax.experimental.pallas import tpu_sc as plsc

N = 10000
E = 320000
D = 128
H = 8
DH = D // H
R = 50
CUT_UP = 5.0
EPS = 1e-5

NC = 2
NS = 16
NW = NC * NS
EPT = E // NW

BA = 40
JA = EPT // BA
BB = 80
JB = EPT // BB

NPAD = 10240
ROWS = NPAD // NS
ZR = 32

_NB = 1000
_EB = 4000

_SC_PARAMS = pltpu.CompilerParams(use_tc_tiling_on_sc=False,
                                  needs_layout_passes=False)


def _silu(x):
    return x * (1.0 / (1.0 + jnp.exp(-x)))


def _node_body(x_ref, vec_ref, ln_w_ref, ln_b_ref, Wq_ref, bq_ref, Wk_ref,
               bk_ref, Wv_ref, bv_ref, Wvec_ref, bvec_ref,
               q_ref, t_ref, vdot_ref, vec3_ref):
    x = x_ref[...]
    mu = jnp.mean(x, axis=-1, keepdims=True)
    var = jnp.mean((x - mu) ** 2, axis=-1, keepdims=True)
    xn = (x - mu) / jnp.sqrt(var + EPS) * ln_w_ref[...] + ln_b_ref[...]
    q_ref[...] = xn @ Wq_ref[...] + bq_ref[...]
    t_ref[:, 0:D] = xn @ Wk_ref[...] + bk_ref[...]
    t_ref[:, D:4 * D] = xn @ Wv_ref[...] + bv_ref[...]
    Wvec = Wvec_ref[...]
    bvec = bvec_ref[...]
    vdot = jnp.zeros((_NB, D), jnp.float32)
    for i in range(3):
        vi = vec_ref[:, i, :]
        t_ref[:, 4 * D + i * D:4 * D + (i + 1) * D] = vi
        vp = vi @ Wvec + bvec
        vdot = vdot + vp[:, :D] * vp[:, D:2 * D]
        vec3_ref[:, i, :] = vp[:, 2 * D:]
    vdot_ref[...] = vdot


def _edge_body(f_ref, r_ref, d_ref, Wdk_ref, bdk_ref, Wdv_ref, bdv_ref,
               ef_ref, mt_ref):
    f = f_ref[...]
    ef_ref[:, 0:D] = _silu(f @ Wdk_ref[...] + bdk_ref[...])
    ef_ref[:, D:4 * D] = _silu(f @ Wdv_ref[...] + bdv_ref[...])
    r = r_ref[...]
    cut = 0.5 * (jnp.cos(r * jnp.pi / CUT_UP) + 1.0) * (r < CUT_UP).astype(r.dtype)
    col = lax.broadcasted_iota(jnp.int32, (_EB, 64), 1) // 16
    d = d_ref[...]
    m = cut * (col == 0).astype(jnp.float32)
    for i in range(3):
        m = m + d[:, i:i + 1] * (col == i + 1).astype(jnp.float32)
    mt_ref[...] = m


def _scA_body(ei_hbm, q_hbm, t_hbm, ef_hbm, mt_hbm,
              msg_hbm,
              sid, did, qb, tb, eb, mb, sp, semI, semG, semS):
    c = lax.axis_index("c")
    s = lax.axis_index("s")
    wid = s * NC + c
    tbase = wid * EPT

    src_hbm = ei_hbm.at[0]
    dst_hbm = ei_hbm.at[1]
    pltpu.async_copy(src_hbm.at[pl.ds(tbase, BA)], sid, semI)
    pltpu.async_copy(dst_hbm.at[pl.ds(tbase, BA)], did, semI)

    def blockA(j, _):
        base = tbase + j * BA
        pltpu.make_async_copy(src_hbm.at[pl.ds(base, BA)], sid, semI).wait()
        pltpu.make_async_copy(dst_hbm.at[pl.ds(base, BA)], did, semI).wait()
        hg = [pltpu.async_copy(t_hbm.at[sid], tb, semG),
              pltpu.async_copy(q_hbm.at[did], qb, semG),
              pltpu.async_copy(ef_hbm.at[pl.ds(base, BA), :], eb, semG),
              pltpu.async_copy(mt_hbm.at[pl.ds(base, BA), :], mb, semG)]
        for h in hg:
            h.wait()

        @pl.when(j + 1 < JA)
        def _():
            nbase = tbase + (j + 1) * BA
            pltpu.async_copy(src_hbm.at[pl.ds(nbase, BA)], sid, semI)
            pltpu.async_copy(dst_hbm.at[pl.ds(nbase, BA)], did, semI)

        @pl.when(j > 0)
        def _():
            for g in range(5):
                pltpu.make_async_copy(
                    msg_hbm.at[g, pl.ds(base, BA), :], sp.at[g], semS).wait()

        @plsc.parallel_loop(0, BA, unroll=4)
        def edge(e):
            dvs = tuple(mb[e, pl.ds(16 * (i + 1), 16)] for i in range(3))
            for h in range(H):
                sp[0, e, pl.ds(h * 16, 16)] = (qb[e, pl.ds(h * 16, 16)]
                                               * tb[e, pl.ds(h * 16, 16)]
                                               * eb[e, pl.ds(h * 16, 16)])
                vo = D + h * 48
                sp[1, e, pl.ds(h * 16, 16)] = (tb[e, pl.ds(vo, 16)]
                                               * eb[e, pl.ds(vo, 16)])
                v1 = tb[e, pl.ds(vo + 16, 16)] * eb[e, pl.ds(vo + 16, 16)]
                v2 = tb[e, pl.ds(vo + 32, 16)] * eb[e, pl.ds(vo + 32, 16)]
                for i in range(3):
                    vv = tb[e, pl.ds(4 * D + i * D + h * 16, 16)]
                    sp[2 + i, e, pl.ds(h * 16, 16)] = vv * v1 + v2 * dvs[i]

        for g in range(5):
            pltpu.async_copy(sp.at[g], msg_hbm.at[g, pl.ds(base, BA), :], semS)
        return 0
    lax.fori_loop(0, JA, blockA, 0)
    for g in range(5):
        pltpu.make_async_copy(
            msg_hbm.at[g, pl.ds(tbase, BA), :], sp.at[g], semS).wait()


_scA_kernel = pl.kernel(
    _scA_body,
    out_type=[jax.ShapeDtypeStruct((5, E, D), jnp.float32)],
    mesh=plsc.VectorSubcoreMesh(core_axis_name="c", subcore_axis_name="s"),
    compiler_params=_SC_PARAMS,
    scratch_types=[
        pltpu.VMEM((BA,), jnp.int32),
        pltpu.VMEM((BA,), jnp.int32),
        pltpu.VMEM((BA, D), jnp.float32),
        pltpu.VMEM((BA, 7 * D), jnp.float32),
        pltpu.VMEM((BA, 4 * D), jnp.float32),
        pltpu.VMEM((BA, 64), jnp.float32),
        pltpu.VMEM((5, BA, D), jnp.float32),
        pltpu.SemaphoreType.DMA,
        pltpu.SemaphoreType.DMA,
        pltpu.SemaphoreType.DMA,
    ],
)


def _scB_body(ei_hbm, xm_hbm, msg_hbm, px_hbm, pv_hbm,
              did, rbuf, zb, acc, sem):
    dst_hbm = ei_hbm.at[1]
    c = lax.axis_index("c")
    s = lax.axis_index("s")
    wid = s * NC + c
    tbase = wid * EPT

    def _zrow(i, _):
        for l in range(D // 16):
            zb[i, pl.ds(l * 16, 16)] = jnp.zeros((16,), jnp.float32)
        return 0
    lax.fori_loop(0, ZR, _zrow, 0)

    def _zero_acc():
        for rep in range(ROWS // ZR):
            pltpu.sync_copy(zb, acc.at[pl.ds(s * ROWS + rep * ZR, ZR)])

    for p in range(4):
        _zero_acc()
        plsc.subcore_barrier()

        def blockB(j, _, p=p):
            base = tbase + j * BB
            h1 = pltpu.async_copy(dst_hbm.at[pl.ds(base, BB)], did, sem)
            if p == 0:
                h2 = pltpu.async_copy(xm_hbm.at[pl.ds(base, BB), :], rbuf, sem)
            else:
                h2 = pltpu.async_copy(
                    msg_hbm.at[p + 1, pl.ds(base, BB), :], rbuf, sem)
            h1.wait()
            h2.wait()
            pltpu.sync_copy(rbuf, acc.at[did], add=True)
            return 0
        lax.fori_loop(0, JB, blockB, 0)
        plsc.subcore_barrier()
        if p == 0:
            pltpu.sync_copy(acc.at[pl.ds(s * ROWS, ROWS)],
                            px_hbm.at[c, pl.ds(s * ROWS, ROWS), :])
        else:
            pltpu.sync_copy(acc.at[pl.ds(s * ROWS, ROWS)],
                            pv_hbm.at[p - 1, c, pl.ds(s * ROWS, ROWS), :])
        plsc.subcore_barrier()


_scB_kernel = pl.kernel(
    _scB_body,
    out_type=[
        jax.ShapeDtypeStruct((NC, NPAD, D), jnp.float32),
        jax.ShapeDtypeStruct((3, NC, NPAD, D), jnp.float32),
    ],
    mesh=plsc.VectorSubcoreMesh(core_axis_name="c", subcore_axis_name="s"),
    compiler_params=_SC_PARAMS,
    scratch_types=[
        pltpu.VMEM((BB,), jnp.int32),
        pltpu.VMEM((BB, D), jnp.float32),
        pltpu.VMEM((ZR, D), jnp.float32),
        pltpu.VMEM_SHARED((NPAD, D), jnp.float32),
        pltpu.SemaphoreType.DMA,
    ],
)


def _attn_body(p_ref, xraw_ref, r_ref, dx_ref):
    rows = lax.broadcasted_iota(jnp.int32, (D, H), 0) // DH
    cols = lax.broadcasted_iota(jnp.int32, (D, H), 1)
    S = (rows == cols).astype(jnp.float32)
    a = p_ref[0] @ S
    r = r_ref[...]
    cut = 0.5 * (jnp.cos(r * jnp.pi / CUT_UP) + 1.0) * (r < CUT_UP).astype(r.dtype)
    a = _silu(a) * cut
    dx_ref[...] = xraw_ref[0] * (a @ S.T)


def _final_body(px_ref, pv_ref, vdot_ref, vec3_ref, Wo_ref, bo_ref,
                dx_ref, dvec_ref):
    xagg = px_ref[0] + px_ref[1]
    o = xagg @ Wo_ref[...] + bo_ref[...]
    o1 = o[:, :D]
    o2 = o[:, D:2 * D]
    o3 = o[:, 2 * D:]
    dx_ref[...] = vdot_ref[...] * o2 + o3
    for i in range(3):
        dvec_ref[:, i, :] = (vec3_ref[:, i, :] * o1
                             + pv_ref[i, 0] + pv_ref[i, 1])


def kernel(x, vec, edge_index, r_ij, f_ij, d_ij, ln_w, ln_b, Wq, bq, Wk, bk,
           Wv, bv, Wo, bo, Wvec, bvec, Wdk, bdk, Wdv, bdv):
    nw = lambda shp: pl.BlockSpec(shp, lambda i: (0,) * len(shp))
    qtab, tsrc, vdot, vec3 = pl.pallas_call(
        _node_body,
        grid=(N // _NB,),
        in_specs=[
            pl.BlockSpec((_NB, D), lambda i: (i, 0)),
            pl.BlockSpec((_NB, 3, D), lambda i: (i, 0, 0)),
            nw((D,)), nw((D,)),
            nw((D, D)), nw((D,)),
            nw((D, D)), nw((D,)),
            nw((D, 3 * D)), nw((3 * D,)),
            nw((D, 3 * D)), nw((3 * D,)),
        ],
        out_specs=[
            pl.BlockSpec((_NB, D), lambda i: (i, 0)),
            pl.BlockSpec((_NB, 7 * D), lambda i: (i, 0)),
            pl.BlockSpec((_NB, D), lambda i: (i, 0)),
            pl.BlockSpec((_NB, 3, D), lambda i: (i, 0, 0)),
        ],
        out_shape=[
            jax.ShapeDtypeStruct((N, D), jnp.float32),
            jax.ShapeDtypeStruct((N, 7 * D), jnp.float32),
            jax.ShapeDtypeStruct((N, D), jnp.float32),
            jax.ShapeDtypeStruct((N, 3, D), jnp.float32),
        ],
    )(x, vec, ln_w, ln_b, Wq, bq, Wk, bk, Wv, bv, Wvec, bvec)

    ef, meta = pl.pallas_call(
        _edge_body,
        grid=(E // _EB,),
        in_specs=[
            pl.BlockSpec((_EB, R), lambda i: (i, 0)),
            pl.BlockSpec((_EB, 1), lambda i: (i, 0)),
            pl.BlockSpec((_EB, 3), lambda i: (i, 0)),
            nw((R, D)), nw((D,)),
            nw((R, 3 * D)), nw((3 * D,)),
        ],
        out_specs=[
            pl.BlockSpec((_EB, 4 * D), lambda i: (i, 0)),
            pl.BlockSpec((_EB, 64), lambda i: (i, 0)),
        ],
        out_shape=[
            jax.ShapeDtypeStruct((E, 4 * D), jnp.float32),
            jax.ShapeDtypeStruct((E, 64), jnp.float32),
        ],
    )(f_ij, r_ij.reshape(E, 1), d_ij, Wdk, bdk, Wdv, bdv)

    (msg,) = _scA_kernel(edge_index, qtab, tsrc, ef, meta)
    xmsg = pl.pallas_call(
        _attn_body,
        grid=(E // _EB,),
        in_specs=[
            pl.BlockSpec((1, _EB, D), lambda i: (0, i, 0)),
            pl.BlockSpec((1, _EB, D), lambda i: (1, i, 0)),
            pl.BlockSpec((_EB, 1), lambda i: (i, 0)),
        ],
        out_specs=pl.BlockSpec((_EB, D), lambda i: (i, 0)),
        out_shape=jax.ShapeDtypeStruct((E, D), jnp.float32),
    )(msg, msg, r_ij.reshape(E, 1))
    px, pv = _scB_kernel(edge_index, xmsg, msg)

    dx, dvec = pl.pallas_call(
        _final_body,
        grid=(N // _NB,),
        in_specs=[
            pl.BlockSpec((NC, _NB, D), lambda i: (0, i, 0)),
            pl.BlockSpec((3, NC, _NB, D), lambda i: (0, 0, i, 0)),
            pl.BlockSpec((_NB, D), lambda i: (i, 0)),
            pl.BlockSpec((_NB, 3, D), lambda i: (i, 0, 0)),
            nw((D, 3 * D)), nw((3 * D,)),
        ],
        out_specs=[
            pl.BlockSpec((_NB, D), lambda i: (i, 0)),
            pl.BlockSpec((_NB, 3, D), lambda i: (i, 0, 0)),
        ],
        out_shape=[
            jax.ShapeDtypeStruct((N, D), jnp.float32),
            jax.ShapeDtypeStruct((N, 3, D), jnp.float32),
        ],
    )(px, pv, vdot, vec3, Wo, bo)
    return (dx, dvec)
